# Initial kernel scaffold; baseline (speedup 1.0000x reference)
import jax
import jax.numpy as jnp
from jax.experimental import pallas as pl


def kernel(x_atoms, edge_index, edge_attr, frag_index, x_frags, atom_to_frag_ids, W_atom, b_atom, W_edge, b_edge, W_f1, b_f1, W_f2, b_f2):
    raise NotImplementedError("write your pallas kernel here")


# speedup vs baseline: 21.0228x; 21.0228x over previous
import functools

import jax
import jax.numpy as jnp
from jax import lax
from jax.experimental import pallas as pl
from jax.experimental.pallas import tpu as pltpu
from jax.experimental.pallas import tpu_sc as plsc

NA = 10000
NAP = 10240
NE = 320000
ECH = 80
EPW = ECH * 128
NF = 2000
NFP = 2048
NFE = 8000
FCH = 2
D = 128
NC, NS = 2, 16
NW = NC * NS
APT = NAP // NS
APW = NAP // NW
FPT = NFP // NS

_MESH = plsc.VectorSubcoreMesh(core_axis_name="c", subcore_axis_name="s",
                               num_cores=NC, num_subcores=NS)

_f32 = jnp.float32


def _zero_fill(zrow, nrows):
    def body(r, _):
        for c in range(D // 16):
            zrow[r, pl.ds(c * 16, 16)] = jnp.zeros((16,), _f32)
        return 0
    lax.fori_loop(0, nrows, body, 0)


@functools.partial(
    pl.kernel, mesh=_MESH,
    out_type=jax.ShapeDtypeStruct((NC, NAP), _f32),
    scratch_types=[
        pltpu.VMEM((ECH, 128), jnp.int32),
        pltpu.VMEM((128,), _f32),
        pltpu.VMEM((APT,), _f32),
        pltpu.VMEM_SHARED((NAP,), _f32),
    ],
)
def _sc_degree(srcd_hbm, degp_hbm, idx_v, ones_v, zbuf, deg_sp):
    cid = lax.axis_index("c")
    sid = lax.axis_index("s")
    w = cid * NS + sid
    for c in range(128 // 16):
        ones_v[pl.ds(c * 16, 16)] = jnp.ones((16,), _f32)
    def zb(i, _):
        zbuf[pl.ds(i * 16, 16)] = jnp.zeros((16,), _f32)
        return 0
    lax.fori_loop(0, APT // 16, zb, 0)
    pltpu.sync_copy(zbuf, deg_sp.at[pl.ds(sid * APT, APT)])
    plsc.subcore_barrier()
    pltpu.sync_copy(srcd_hbm.at[w], idx_v)
    def body(j, _):
        pltpu.sync_copy(ones_v, deg_sp.at[idx_v.at[j]], add=True)
        return 0
    lax.fori_loop(0, ECH, body, 0)
    plsc.subcore_barrier()
    pltpu.sync_copy(deg_sp.at[pl.ds(sid * APT, APT)],
                    degp_hbm.at[cid, pl.ds(sid * APT, APT)])


HALF = NAP // NC
ACCR = HALF + 64
ECH2 = (NW * EPW) // (NS * 128)


@functools.partial(
    pl.kernel, mesh=_MESH,
    out_type=jax.ShapeDtypeStruct((NAP, D), _f32),
    scratch_types=[
        pltpu.VMEM((ECH2, 128), jnp.int32),
        pltpu.VMEM((ECH2, 128), jnp.int32),
        pltpu.VMEM((2, 128), jnp.int32),
        pltpu.VMEM((2, 128, D), _f32),
        pltpu.VMEM((64, D), _f32),
        pltpu.VMEM_SHARED((ACCR, D), _f32),
        pltpu.SemaphoreType.DMA,
        pltpu.SemaphoreType.DMA,
    ],
)
def _sc_edges(g_hbm, srcg_hbm, dstp_hbm, acc_hbm,
              src_v, dst_v, dl, rb, zrow, acc_sp, sem0, sem1):
    cid = lax.axis_index("c")
    sid = lax.axis_index("s")
    _zero_fill(zrow, 64)
    zpt = ACCR // NS
    for k in range(zpt // 64):
        pltpu.sync_copy(zrow, acc_sp.at[pl.ds(sid * zpt + k * 64, 64)])
    if zpt % 64:
        pltpu.sync_copy(zrow.at[pl.ds(0, zpt % 64)],
                        acc_sp.at[pl.ds(sid * zpt + (zpt // 64) * 64,
                                        zpt % 64)])
    pltpu.sync_copy(srcg_hbm.at[sid], src_v)
    pltpu.sync_copy(dstp_hbm.at[sid], dst_v)
    plsc.subcore_barrier()
    base_lo = cid * HALF

    def remap(j, b):
        for c in range(128 // 16):
            s = pl.ds(c * 16, 16)
            d = dst_v[j, s]
            local = d - base_lo
            ok = (local >= 0) & (local < HALF)
            dum = HALF + jnp.bitwise_and(d, 63)
            dl[b, s] = jnp.where(ok, local, dum)

    def body(k, _):
        j0 = 2 * k
        j1 = 2 * k + 1
        c0 = pltpu.async_copy(g_hbm.at[src_v.at[j0]], rb.at[0], sem0)
        c1 = pltpu.async_copy(g_hbm.at[src_v.at[j1]], rb.at[1], sem1)
        remap(j0, 0)
        remap(j1, 1)
        c0.wait()
        pltpu.sync_copy(rb.at[0], acc_sp.at[dl.at[0]], add=True)
        c1.wait()
        pltpu.sync_copy(rb.at[1], acc_sp.at[dl.at[1]], add=True)
        return 0
    lax.fori_loop(0, ECH2 // 2, body, 0)
    plsc.subcore_barrier()
    dpt = HALF // NS
    pltpu.sync_copy(acc_sp.at[pl.ds(sid * dpt, dpt)],
                    acc_hbm.at[pl.ds(cid * HALF + sid * dpt, dpt)])


@functools.partial(
    pl.kernel, mesh=_MESH,
    out_type=(jax.ShapeDtypeStruct((NAP, D), _f32),
              jax.ShapeDtypeStruct((NFP, D), _f32),
              jax.ShapeDtypeStruct((NFP, D), _f32)),
    scratch_types=[
        pltpu.VMEM((64, D), _f32),
        pltpu.VMEM((64, D), _f32),
        pltpu.VMEM((64, D), _f32),
        pltpu.VMEM((64, D), _f32),
        pltpu.VMEM((APW // 64, 64), jnp.int32),
        pltpu.VMEM_SHARED((NFP, D), _f32),
        pltpu.SemaphoreType.DMA,
    ],
)
def _sc_combine_pool(acc_hbm, g_hbm, dm_hbm, a2f_hbm, xnew_hbm,
                     fragp0_hbm, fragp1_hbm,
                     b0, bg, bd, xb, ids_v, frag_sp, sem):
    cid = lax.axis_index("c")
    sid = lax.axis_index("s")
    w = cid * NS + sid
    base = w * APW
    _zero_fill(xb, 64)
    for k in range(FPT // 64):
        pltpu.sync_copy(xb, frag_sp.at[pl.ds(sid * FPT + k * 64, 64)])
    pltpu.sync_copy(a2f_hbm.at[w], ids_v)
    plsc.subcore_barrier()
    for j in range(APW // 64):
        r0 = base + j * 64
        c0 = pltpu.async_copy(acc_hbm.at[pl.ds(r0, 64)], b0, sem)
        c2 = pltpu.async_copy(g_hbm.at[pl.ds(r0, 64)], bg, sem)
        c3 = pltpu.async_copy(dm_hbm.at[pl.ds(r0, 64)], bd, sem)
        c0.wait(); c2.wait(); c3.wait()
        def rows(r, _):
            for c in range(D // 16):
                s = pl.ds(c * 16, 16)
                xb[r, s] = (b0[r, s] + bg[r, s]) * bd[r, s]
            return 0
        lax.fori_loop(0, 64, rows, 0)
        pltpu.sync_copy(xb, xnew_hbm.at[pl.ds(r0, 64)])
        pltpu.sync_copy(xb, frag_sp.at[ids_v.at[j]], add=True)
    plsc.subcore_barrier()

    @pl.when(cid == 0)
    def _dump0():
        pltpu.sync_copy(frag_sp.at[pl.ds(sid * FPT, FPT)],
                        fragp0_hbm.at[pl.ds(sid * FPT, FPT)])

    @pl.when(cid == 1)
    def _dump1():
        pltpu.sync_copy(frag_sp.at[pl.ds(sid * FPT, FPT)],
                        fragp1_hbm.at[pl.ds(sid * FPT, FPT)])


@functools.partial(
    pl.kernel, mesh=_MESH,
    out_type=jax.ShapeDtypeStruct((NC, NFP, D), _f32),
    scratch_types=[
        pltpu.VMEM((128, D), _f32),
        pltpu.VMEM((128, D), _f32),
        pltpu.VMEM((FCH, 128), jnp.int32),
        pltpu.VMEM((FCH, 128), jnp.int32),
        pltpu.VMEM_SHARED((NFP, D), _f32),
        pltpu.SemaphoreType.DMA,
    ],
)
def _sc_frag(fragp0_hbm, fragp1_hbm, fsrc_hbm, ftgt_hbm, ffsp_hbm,
             fa, fb, sids, tids, ffs_sp, sem):
    cid = lax.axis_index("c")
    sid = lax.axis_index("s")
    w = cid * NS + sid
    r0 = sid * FPT
    _zero_fill(fb, 128)
    pltpu.sync_copy(fb, ffs_sp.at[pl.ds(r0, FPT)])
    pltpu.sync_copy(fsrc_hbm.at[w], sids)
    pltpu.sync_copy(ftgt_hbm.at[w], tids)
    plsc.subcore_barrier()
    for j in range(FCH):
        c0 = pltpu.async_copy(fragp0_hbm.at[sids.at[j]], fa, sem)
        c1 = pltpu.async_copy(fragp1_hbm.at[sids.at[j]], fb, sem)
        c0.wait(); c1.wait()
        def rows(r, _):
            for c in range(D // 16):
                s = pl.ds(c * 16, 16)
                fa[r, s] = fa[r, s] + fb[r, s]
            return 0
        lax.fori_loop(0, 128, rows, 0)
        pltpu.sync_copy(fa, ffs_sp.at[tids.at[j]], add=True)
    plsc.subcore_barrier()
    pltpu.sync_copy(ffs_sp.at[pl.ds(r0, FPT)],
                    ffsp_hbm.at[cid, pl.ds(r0, FPT)])


def _tc_embed_body(x_ref, dinv_ref, wT_ref, b_ref, g_ref, dm_ref):
    h = jnp.dot(x_ref[...], wT_ref[...],
                preferred_element_type=_f32) + b_ref[...]
    dinv = dinv_ref[...]
    g_ref[...] = dinv * h
    dm_ref[...] = jnp.broadcast_to(dinv, (NAP, D))


_tc_embed = pl.pallas_call(
    _tc_embed_body,
    out_shape=(jax.ShapeDtypeStruct((NAP, D), _f32),
               jax.ShapeDtypeStruct((NAP, D), _f32)),
)


def _tc_mlp_body(ffsp_ref, w1T_ref, b1_ref, w2T_ref, b2_ref, out_ref):
    ffs = ffsp_ref[0] + ffsp_ref[1]
    h = jnp.maximum(jnp.dot(ffs, w1T_ref[...],
                            preferred_element_type=_f32) + b1_ref[...], 0.0)
    out_ref[...] = jnp.dot(h, w2T_ref[...],
                           preferred_element_type=_f32) + b2_ref[...]


_tc_mlp = pl.pallas_call(
    _tc_mlp_body,
    out_shape=jax.ShapeDtypeStruct((NFP, D), _f32),
)


def kernel(x_atoms, edge_index, edge_attr, frag_index, x_frags,
           atom_to_frag_ids, W_atom, b_atom, W_edge, b_edge,
           W_f1, b_f1, W_f2, b_f2):
    ei = edge_index.astype(jnp.int32)
    src, dst = ei[0], ei[1]
    npad = NW * EPW - NE
    ar = jnp.arange(npad, dtype=jnp.int32)
    dummy_atom = NA + ar % (NAP - NA)
    src_deg = jnp.concatenate([src, dummy_atom]).reshape(NW, ECH, 128)
    src_gat = jnp.concatenate([src, ar % NA]).reshape(NS, ECH2, 128)
    dst_pad = jnp.concatenate([dst, dummy_atom]).reshape(NS, ECH2, 128)

    a2f = atom_to_frag_ids.astype(jnp.int32)
    arf = jnp.arange(NAP - NA, dtype=jnp.int32)
    a2f_pad = jnp.concatenate([a2f, NF + arf % (NFP - NF)]).reshape(
        NW, APW // 64, 64)

    fi = frag_index.astype(jnp.int32)
    fpad = NW * FCH * 128 - NFE
    arf2 = jnp.arange(fpad, dtype=jnp.int32)
    dummy_frag = NF + arf2 % (NFP - NF)
    fsrc_pad = jnp.concatenate([fi[0], dummy_frag]).reshape(NW, FCH, 128)
    ftgt_pad = jnp.concatenate([fi[1], dummy_frag]).reshape(NW, FCH, 128)

    x_pad = jnp.pad(x_atoms, ((0, NAP - NA), (0, 0)))

    degp = _sc_degree(src_deg)
    dinv_col = lax.rsqrt(degp[0] + degp[1] + 1.0).reshape(NAP, 1)
    g, dm = _tc_embed(x_pad, dinv_col, W_atom.T, b_atom.reshape(1, D))
    acc = _sc_edges(g, src_gat, dst_pad)
    xnew_pad, fragp0, fragp1 = _sc_combine_pool(acc, g, dm, a2f_pad)
    ffsp = _sc_frag(fragp0, fragp1, fsrc_pad, ftgt_pad)
    xfrags = _tc_mlp(ffsp, W_f1.T, b_f1.reshape(1, 2 * D),
                     W_f2.T, b_f2.reshape(1, D))
    return xnew_pad[:NA], xfrags[:NF]


# speedup vs baseline: 21.6073x; 1.0278x over previous
import functools

import jax
import jax.numpy as jnp
from jax import lax
from jax.experimental import pallas as pl
from jax.experimental.pallas import tpu as pltpu
from jax.experimental.pallas import tpu_sc as plsc

NA = 10000
NAP = 10240
NE = 320000
ECH = 80
EPW = ECH * 128
NF = 2000
NFP = 2048
NFE = 8000
FCH = 2
D = 128
NC, NS = 2, 16
NW = NC * NS
APT = NAP // NS
APW = NAP // NW
FPT = NFP // NS

_MESH = plsc.VectorSubcoreMesh(core_axis_name="c", subcore_axis_name="s",
                               num_cores=NC, num_subcores=NS)

_f32 = jnp.float32


def _zero_fill(zrow, nrows):
    def body(r, _):
        for c in range(D // 16):
            zrow[r, pl.ds(c * 16, 16)] = jnp.zeros((16,), _f32)
        return 0
    lax.fori_loop(0, nrows, body, 0)


@functools.partial(
    pl.kernel, mesh=_MESH,
    out_type=jax.ShapeDtypeStruct((NC, NAP), _f32),
    scratch_types=[
        pltpu.VMEM((ECH, 128), jnp.int32),
        pltpu.VMEM((128,), _f32),
        pltpu.VMEM((APT,), _f32),
        pltpu.VMEM_SHARED((NAP,), _f32),
    ],
)
def _sc_degree(srcd_hbm, degp_hbm, idx_v, ones_v, zbuf, deg_sp):
    cid = lax.axis_index("c")
    sid = lax.axis_index("s")
    w = cid * NS + sid
    for c in range(128 // 16):
        ones_v[pl.ds(c * 16, 16)] = jnp.ones((16,), _f32)
    def zb(i, _):
        zbuf[pl.ds(i * 16, 16)] = jnp.zeros((16,), _f32)
        return 0
    lax.fori_loop(0, APT // 16, zb, 0)
    pltpu.sync_copy(zbuf, deg_sp.at[pl.ds(sid * APT, APT)])
    plsc.subcore_barrier()
    pltpu.sync_copy(srcd_hbm.at[w], idx_v)
    def body(j, _):
        pltpu.sync_copy(ones_v, deg_sp.at[idx_v.at[j]], add=True)
        return 0
    lax.fori_loop(0, ECH, body, 0)
    plsc.subcore_barrier()
    pltpu.sync_copy(deg_sp.at[pl.ds(sid * APT, APT)],
                    degp_hbm.at[cid, pl.ds(sid * APT, APT)])


HALF = NAP // NC
ACCR = HALF + 64
ECH2 = (NW * EPW) // (NS * 128)
NB = 2
DPT = HALF // NS


@functools.partial(
    pl.kernel, mesh=_MESH,
    out_type=(jax.ShapeDtypeStruct((NAP, D), _f32),
              jax.ShapeDtypeStruct((NFP, D), _f32),
              jax.ShapeDtypeStruct((NFP, D), _f32)),
    scratch_types=(
        [pltpu.VMEM((ECH2, 128), jnp.int32),
         pltpu.VMEM((ECH2, 128), jnp.int32),
         pltpu.VMEM((NB, 128, D), _f32),
         pltpu.VMEM((DPT // 64, 64), jnp.int32),
         pltpu.VMEM_SHARED((ACCR, D), _f32)]
        + [pltpu.SemaphoreType.DMA] * (2 * NB + 1)),
)
def _sc_edges_pool(g_hbm, dm_hbm, a2f2_hbm, srcg_hbm, dstp_hbm,
                   xnew_hbm, fragp0_hbm, fragp1_hbm,
                   src_v, dst_v, rb, ids_v, acc_sp, *sems):
    gsem = sems[:NB]
    ssem = sems[NB:2 * NB]
    esem = sems[2 * NB]
    cid = lax.axis_index("c")
    sid = lax.axis_index("s")

    def zb(r, _):
        for c in range(D // 16):
            rb[0, r, pl.ds(c * 16, 16)] = jnp.zeros((16,), _f32)
        return 0
    lax.fori_loop(0, 64, zb, 0)
    zpt = ACCR // NS
    for k in range(zpt // 64):
        pltpu.sync_copy(rb.at[0, pl.ds(0, 64)],
                        acc_sp.at[pl.ds(sid * zpt + k * 64, 64)])
    if zpt % 64:
        pltpu.sync_copy(rb.at[0, pl.ds(0, zpt % 64)],
                        acc_sp.at[pl.ds(sid * zpt + (zpt // 64) * 64,
                                        zpt % 64)])
    pltpu.sync_copy(srcg_hbm.at[sid], src_v)
    pltpu.sync_copy(dstp_hbm.at[sid], dst_v)
    base_g = cid * HALF + sid * DPT
    for j in range(DPT // 64):
        pltpu.sync_copy(a2f2_hbm.at[pl.ds(base_g + j * 64, 64)],
                        ids_v.at[j])

    base_lo = cid * HALF

    def remap_all(j, _):
        for c in range(128 // 16):
            s = pl.ds(c * 16, 16)
            d = dst_v[j, s]
            local = d - base_lo
            ok = (local >= 0) & (local < HALF)
            dst_v[j, s] = jnp.where(ok, local,
                                    HALF + jnp.bitwise_and(d, 63))
        return 0
    lax.fori_loop(0, ECH2, remap_all, 0)
    plsc.subcore_barrier()

    def body(k, _):
        j0 = 2 * k
        j1 = 2 * k + 1

        @pl.when(k > 0)
        def _w0():
            pltpu.make_async_copy(rb.at[0], acc_sp.at[dst_v.at[j0]],
                                  ssem[0]).wait()
        pltpu.async_copy(g_hbm.at[src_v.at[j0]], rb.at[0], gsem[0])

        @pl.when(k > 0)
        def _w1():
            pltpu.make_async_copy(rb.at[1], acc_sp.at[dst_v.at[j1]],
                                  ssem[1]).wait()
        pltpu.async_copy(g_hbm.at[src_v.at[j1]], rb.at[1], gsem[1])

        pltpu.make_async_copy(g_hbm.at[src_v.at[j0]], rb.at[0],
                              gsem[0]).wait()
        pltpu.async_copy(rb.at[0], acc_sp.at[dst_v.at[j0]], ssem[0],
                         add=True)
        pltpu.make_async_copy(g_hbm.at[src_v.at[j1]], rb.at[1],
                              gsem[1]).wait()
        pltpu.async_copy(rb.at[1], acc_sp.at[dst_v.at[j1]], ssem[1],
                         add=True)
        return 0
    lax.fori_loop(0, ECH2 // 2, body, 0)
    pltpu.make_async_copy(rb.at[0], acc_sp.at[dst_v.at[0]], ssem[0]).wait()
    pltpu.make_async_copy(rb.at[1], acc_sp.at[dst_v.at[1]], ssem[1]).wait()
    plsc.subcore_barrier()

    b0 = rb.at[0, pl.ds(0, 64)]
    bg = rb.at[1, pl.ds(0, 64)]
    bd = rb.at[0, pl.ds(64, 64)]
    xb = rb.at[1, pl.ds(64, 64)]
    for j in range(DPT // 64):
        r0 = sid * DPT + j * 64
        rg = base_g + j * 64
        c1 = pltpu.async_copy(g_hbm.at[pl.ds(rg, 64)], bg, esem)
        c2 = pltpu.async_copy(dm_hbm.at[pl.ds(rg, 64)], bd, esem)
        pltpu.sync_copy(acc_sp.at[pl.ds(r0, 64)], b0)
        c1.wait()
        c2.wait()

        def rows(r, _):
            for c in range(D // 16):
                s = pl.ds(c * 16, 16)
                rb[1, 64 + r, s] = ((rb[0, r, s] + rb[1, r, s])
                                    * rb[0, 64 + r, s])
            return 0
        lax.fori_loop(0, 64, rows, 0)
        pltpu.sync_copy(xb, xnew_hbm.at[pl.ds(rg, 64)])
    plsc.subcore_barrier()

    def zb2(r, _):
        for c in range(D // 16):
            rb[0, r, pl.ds(c * 16, 16)] = jnp.zeros((16,), _f32)
        return 0
    lax.fori_loop(0, 64, zb2, 0)
    for k in range(FPT // 64):
        pltpu.sync_copy(rb.at[0, pl.ds(0, 64)],
                        acc_sp.at[pl.ds(sid * FPT + k * 64, 64)])
    plsc.subcore_barrier()
    for j in range(DPT // 64):
        rg = base_g + j * 64
        pltpu.sync_copy(xnew_hbm.at[pl.ds(rg, 64)], bg)
        pltpu.sync_copy(bg, acc_sp.at[ids_v.at[j]], add=True)
    plsc.subcore_barrier()

    @pl.when(cid == 0)
    def _dump0():
        pltpu.sync_copy(acc_sp.at[pl.ds(sid * FPT, FPT)],
                        fragp0_hbm.at[pl.ds(sid * FPT, FPT)])

    @pl.when(cid == 1)
    def _dump1():
        pltpu.sync_copy(acc_sp.at[pl.ds(sid * FPT, FPT)],
                        fragp1_hbm.at[pl.ds(sid * FPT, FPT)])


@functools.partial(
    pl.kernel, mesh=_MESH,
    out_type=jax.ShapeDtypeStruct((NC, NFP, D), _f32),
    scratch_types=[
        pltpu.VMEM((128, D), _f32),
        pltpu.VMEM((128, D), _f32),
        pltpu.VMEM((FCH, 128), jnp.int32),
        pltpu.VMEM((FCH, 128), jnp.int32),
        pltpu.VMEM_SHARED((NFP, D), _f32),
        pltpu.SemaphoreType.DMA,
    ],
)
def _sc_frag(fragp0_hbm, fragp1_hbm, fsrc_hbm, ftgt_hbm, ffsp_hbm,
             fa, fb, sids, tids, ffs_sp, sem):
    cid = lax.axis_index("c")
    sid = lax.axis_index("s")
    w = cid * NS + sid
    r0 = sid * FPT
    _zero_fill(fb, 128)
    pltpu.sync_copy(fb, ffs_sp.at[pl.ds(r0, FPT)])
    pltpu.sync_copy(fsrc_hbm.at[w], sids)
    pltpu.sync_copy(ftgt_hbm.at[w], tids)
    plsc.subcore_barrier()
    for j in range(FCH):
        c0 = pltpu.async_copy(fragp0_hbm.at[sids.at[j]], fa, sem)
        c1 = pltpu.async_copy(fragp1_hbm.at[sids.at[j]], fb, sem)
        c0.wait(); c1.wait()
        def rows(r, _):
            for c in range(D // 16):
                s = pl.ds(c * 16, 16)
                fa[r, s] = fa[r, s] + fb[r, s]
            return 0
        lax.fori_loop(0, 128, rows, 0)
        pltpu.sync_copy(fa, ffs_sp.at[tids.at[j]], add=True)
    plsc.subcore_barrier()
    pltpu.sync_copy(ffs_sp.at[pl.ds(r0, FPT)],
                    ffsp_hbm.at[cid, pl.ds(r0, FPT)])


def _tc_embed_body(x_ref, dinv_ref, wT_ref, b_ref, g_ref, dm_ref):
    h = jnp.dot(x_ref[...], wT_ref[...],
                preferred_element_type=_f32) + b_ref[...]
    dinv = dinv_ref[...]
    g_ref[...] = dinv * h
    dm_ref[...] = jnp.broadcast_to(dinv, (NAP, D))


_tc_embed = pl.pallas_call(
    _tc_embed_body,
    out_shape=(jax.ShapeDtypeStruct((NAP, D), _f32),
               jax.ShapeDtypeStruct((NAP, D), _f32)),
)


def _tc_mlp_body(ffsp_ref, w1T_ref, b1_ref, w2T_ref, b2_ref, out_ref):
    ffs = ffsp_ref[0] + ffsp_ref[1]
    h = jnp.maximum(jnp.dot(ffs, w1T_ref[...],
                            preferred_element_type=_f32) + b1_ref[...], 0.0)
    out_ref[...] = jnp.dot(h, w2T_ref[...],
                           preferred_element_type=_f32) + b2_ref[...]


_tc_mlp = pl.pallas_call(
    _tc_mlp_body,
    out_shape=jax.ShapeDtypeStruct((NFP, D), _f32),
)


def kernel(x_atoms, edge_index, edge_attr, frag_index, x_frags,
           atom_to_frag_ids, W_atom, b_atom, W_edge, b_edge,
           W_f1, b_f1, W_f2, b_f2):
    ei = edge_index.astype(jnp.int32)
    src, dst = ei[0], ei[1]
    npad = NW * EPW - NE
    ar = jnp.arange(npad, dtype=jnp.int32)
    dummy_atom = NA + ar % (NAP - NA)
    src_deg = jnp.concatenate([src, dummy_atom]).reshape(NW, ECH, 128)
    src_gat = jnp.concatenate([src, ar % NA]).reshape(NS, ECH2, 128)
    dst_pad = jnp.concatenate([dst, dummy_atom]).reshape(NS, ECH2, 128)

    a2f = atom_to_frag_ids.astype(jnp.int32)
    arf = jnp.arange(NAP - NA, dtype=jnp.int32)
    a2f_pad = jnp.concatenate([a2f, NF + arf % (NFP - NF)])

    fi = frag_index.astype(jnp.int32)
    fpad = NW * FCH * 128 - NFE
    arf2 = jnp.arange(fpad, dtype=jnp.int32)
    dummy_frag = NF + arf2 % (NFP - NF)
    fsrc_pad = jnp.concatenate([fi[0], dummy_frag]).reshape(NW, FCH, 128)
    ftgt_pad = jnp.concatenate([fi[1], dummy_frag]).reshape(NW, FCH, 128)

    x_pad = jnp.pad(x_atoms, ((0, NAP - NA), (0, 0)))

    degp = _sc_degree(src_deg)
    dinv_col = lax.rsqrt(degp[0] + degp[1] + 1.0).reshape(NAP, 1)
    g, dm = _tc_embed(x_pad, dinv_col, W_atom.T, b_atom.reshape(1, D))
    xnew_pad, fragp0, fragp1 = _sc_edges_pool(g, dm, a2f_pad,
                                              src_gat, dst_pad)
    ffsp = _sc_frag(fragp0, fragp1, fsrc_pad, ftgt_pad)
    xfrags = _tc_mlp(ffsp, W_f1.T, b_f1.reshape(1, 2 * D),
                     W_f2.T, b_f2.reshape(1, D))
    return xnew_pad[:NA], xfrags[:NF]
